# baseline (speedup 1.0000x reference)
import functools

import jax
import jax.numpy as jnp
from jax import lax
from jax.experimental import pallas as pl
from jax.experimental.pallas import tpu as pltpu
from jax.experimental.pallas import tpu_sc as plsc

_NC = 2
_NS = 16
_CHUNK = 112
_G = 8
_NB = 3
_SUP = _G * _NB
_WB = 80


def _make_agg(N_pad, D, E_pad):
  NW = _NC * _NS
  EPW = E_pad // NW
  n_chunks = EPW // _CHUNK
  n_groups = n_chunks // _G
  n_sup = n_chunks // _SUP
  rows_per_tile = N_pad // _NS
  zhops = rows_per_tile // _WB
  mesh = plsc.VectorSubcoreMesh(core_axis_name="c", subcore_axis_name="s")
  out_sds = jax.ShapeDtypeStruct((N_pad, D), jnp.float32)

  @functools.partial(
      pl.kernel,
      mesh=mesh,
      out_type=(out_sds, out_sds),
      scratch_types=[
          pltpu.VMEM((2, _G, _CHUNK), jnp.int32),
          pltpu.VMEM((2, _G, _CHUNK), jnp.int32),
          pltpu.VMEM((_NB, _CHUNK, D), jnp.float32),
          pltpu.VMEM_SHARED((N_pad, D), jnp.float32),
          pltpu.SemaphoreType.DMA,
          pltpu.SemaphoreType.DMA,
          pltpu.SemaphoreType.DMA,
          pltpu.SemaphoreType.DMA,
          pltpu.SemaphoreType.DMA,
          pltpu.SemaphoreType.DMA,
          pltpu.SemaphoreType.DMA,
      ],
  )
  def agg(table_hbm, src_hbm, dst_hbm, zeros_hbm, out0_hbm, out1_hbm,
          src_v, dst_v, rows_v, acc_sh, g0, g1, g2, s0, s1, s2, si):
    sem_g = (g0, g1, g2)
    sem_s = (s0, s1, s2)
    cid = lax.axis_index("c")
    sid = lax.axis_index("s")
    wid = sid * _NC + cid
    row0 = sid * rows_per_tile
    chunk0 = wid * n_chunks

    def g_fire(slot, k, b):
      pltpu.async_copy(table_hbm.at[src_v.at[slot, k]], rows_v.at[b],
                       sem_g[b])

    def g_wait(slot, k, b):
      pltpu.make_async_copy(table_hbm.at[src_v.at[slot, k]], rows_v.at[b],
                            sem_g[b]).wait()

    def s_fire(slot, k, b):
      pltpu.async_copy(rows_v.at[b], acc_sh.at[dst_v.at[slot, k]],
                       sem_s[b], add=True)

    def s_wait(slot, k, b):
      pltpu.make_async_copy(rows_v.at[b], acc_sh.at[dst_v.at[slot, k]],
                            sem_s[b]).wait()

    def idx_fire(grp, slot):
      nxt = chunk0 + grp * _G
      pltpu.async_copy(src_hbm.at[pl.ds(nxt, _G)], src_v.at[slot], si)
      pltpu.async_copy(dst_hbm.at[pl.ds(nxt, _G)], dst_v.at[slot], si)

    def idx_wait(grp, slot):
      nxt = chunk0 + grp * _G
      pltpu.make_async_copy(src_hbm.at[pl.ds(nxt, _G)], src_v.at[slot],
                            si).wait()
      pltpu.make_async_copy(dst_hbm.at[pl.ds(nxt, _G)], dst_v.at[slot],
                            si).wait()

    pltpu.sync_copy(src_hbm.at[pl.ds(chunk0, _G)], src_v.at[0])
    g_fire(0, 0, 0)
    g_fire(0, 1, 1)

    zbuf = rows_v.at[2, pl.ds(0, _WB)]
    pltpu.sync_copy(zeros_hbm, zbuf)
    for z in range(zhops):
      pltpu.async_copy(zbuf, acc_sh.at[pl.ds(row0 + z * _WB, _WB)], s0)
    pltpu.sync_copy(dst_hbm.at[pl.ds(chunk0, _G)], dst_v.at[0])
    for z in range(zhops):
      pltpu.make_async_copy(zbuf, acc_sh.at[pl.ds(row0 + z * _WB, _WB)],
                            s0).wait()
    plsc.subcore_barrier()

    def sup_body(i, carry):
      g3 = 3 * i
      for t in range(_SUP):
        gg, k = divmod(t, _G)
        b = t % _NB
        gslot = lax.rem(g3 + gg, 2)
        g_wait(gslot, k, b)
        s_fire(gslot, k, b)
        if t == 0:
          @pl.when(i >= 1)
          def _():
            s_wait(lax.rem(g3 + 1, 2), _G - 1, (t - 1) % _NB)
        else:
          gq, kq = divmod(t - 1, _G)
          s_wait(lax.rem(g3 + gq, 2), kq, (t - 1) % _NB)
        if k == 0:
          grp = gg + 1
          if grp < _NB:
            idx_fire(g3 + grp, lax.rem(g3 + grp, 2))
          else:
            @pl.when(i + 1 < n_sup)
            def _():
              idx_fire(g3 + grp, lax.rem(g3 + grp, 2))
        gg2, k2 = divmod(t + 2, _G)
        b2 = (t + 2) % _NB

        def fire_next(gg2=gg2, k2=k2, b2=b2):
          if k2 == 0:
            idx_wait(g3 + gg2, lax.rem(g3 + gg2, 2))
          g_fire(lax.rem(g3 + gg2, 2), k2, b2)

        if t + 2 < _SUP:
          fire_next()
        else:
          pl.when(i + 1 < n_sup)(fire_next)
      return carry

    lax.fori_loop(0, n_sup, sup_body, 0)

    s_wait((n_groups - 1) % 2, _G - 1, (n_chunks - 1) % _NB)
    plsc.subcore_barrier()

    out_sel = (out0_hbm, out1_hbm)

    def wb_in(z, b):
      pltpu.async_copy(acc_sh.at[pl.ds(row0 + z * _WB, _WB)],
                       rows_v.at[b, pl.ds(0, _WB)], sem_g[b])

    def wb_wait_in(z, b):
      pltpu.make_async_copy(acc_sh.at[pl.ds(row0 + z * _WB, _WB)],
                            rows_v.at[b, pl.ds(0, _WB)], sem_g[b]).wait()

    def wb_out(z, b):
      r = row0 + z * _WB

      @pl.when(cid == 0)
      def _():
        pltpu.async_copy(rows_v.at[b, pl.ds(0, _WB)],
                         out0_hbm.at[pl.ds(r, _WB)], sem_s[b])

      @pl.when(cid != 0)
      def _():
        pltpu.async_copy(rows_v.at[b, pl.ds(0, _WB)],
                         out1_hbm.at[pl.ds(r, _WB)], sem_s[b])

    def wb_wait_out(z, b):
      r = row0 + z * _WB

      @pl.when(cid == 0)
      def _():
        pltpu.make_async_copy(rows_v.at[b, pl.ds(0, _WB)],
                              out0_hbm.at[pl.ds(r, _WB)], sem_s[b]).wait()

      @pl.when(cid != 0)
      def _():
        pltpu.make_async_copy(rows_v.at[b, pl.ds(0, _WB)],
                              out1_hbm.at[pl.ds(r, _WB)], sem_s[b]).wait()

    wb_in(0, 0)
    for z in range(zhops):
      b = z % 2
      wb_wait_in(z, b)
      wb_out(z, b)
      if z + 1 < zhops:
        if z >= 1:
          wb_wait_out(z - 1, 1 - b)
        wb_in(z + 1, 1 - b)
    for z in (zhops - 2, zhops - 1):
      wb_wait_out(z, z % 2)

  return agg


def _dense(base, p0, p1, w_t, b, n_out):
  D = base.shape[1]
  H = w_t.shape[1]
  BM = 400
  grid = n_out // BM

  def body(base_ref, p0_ref, p1_ref, w_ref, b_ref, o_ref):
    h = base_ref[...] + p0_ref[...] + p1_ref[...]
    acc = jnp.dot(h, w_ref[...], preferred_element_type=jnp.float32)
    acc = acc + b_ref[...]
    o_ref[...] = jnp.where(acc > 0, acc, jnp.exp(acc) - 1.0)

  return pl.pallas_call(
      body,
      grid=(grid,),
      in_specs=[
          pl.BlockSpec((BM, D), lambda i: (i, 0)),
          pl.BlockSpec((BM, D), lambda i: (i, 0)),
          pl.BlockSpec((BM, D), lambda i: (i, 0)),
          pl.BlockSpec((D, H), lambda i: (0, 0)),
          pl.BlockSpec((1, H), lambda i: (0, 0)),
      ],
      out_specs=pl.BlockSpec((BM, H), lambda i: (i, 0)),
      out_shape=jax.ShapeDtypeStruct((n_out, H), jnp.float32),
  )(base, p0, p1, w_t, b.reshape(1, H))


def kernel(features, edge_index, W1, b1, W2, b2):
  N, D = features.shape
  E = edge_index.shape[1]
  H = W1.shape[0]
  NW = _NC * _NS

  N_pad = ((N + 8 + 255) // 256) * 256
  step = NW * _CHUNK * _SUP
  E_pad = ((E + step - 1) // step) * step

  src, dst = edge_index[0], edge_index[1]
  pad_e = E_pad - E
  if E % NW == 0 and pad_e % NW == 0:
    ppt = pad_e // NW
    cyc = jnp.arange(ppt, dtype=jnp.int32) % (N_pad - N)
    pad_blk = jnp.broadcast_to(cyc, (NW, ppt))
    src_p = jnp.concatenate(
        [src.reshape(NW, E // NW), pad_blk], axis=1).reshape(-1, _CHUNK)
    dst_p = jnp.concatenate(
        [dst.reshape(NW, E // NW), N + pad_blk], axis=1).reshape(-1, _CHUNK)
  else:
    cyc = jnp.arange(pad_e, dtype=jnp.int32) % (N_pad - N)
    src_p = jnp.concatenate([src, cyc]).reshape(-1, _CHUNK)
    dst_p = jnp.concatenate([dst, N + cyc]).reshape(-1, _CHUNK)
  zeros_rows = jnp.zeros((_WB, D), jnp.float32)

  agg1 = _make_agg(N_pad, D, E_pad)
  p0, p1 = agg1(features, src_p, dst_p, zeros_rows)
  x = _dense(features, p0, p1, W1.T, b1, N)

  agg2 = _make_agg(N_pad, H, E_pad)
  q0, q1 = agg2(x, src_p, dst_p, zeros_rows)
  return _dense(x, q0, q1, W2.T, b2, N)


# speedup vs baseline: 1.0955x; 1.0955x over previous
import functools

import jax
import jax.numpy as jnp
from jax import lax
from jax.experimental import pallas as pl
from jax.experimental.pallas import tpu as pltpu
from jax.experimental.pallas import tpu_sc as plsc

_NC = 2
_NS = 16
_CHUNK = 128
_G = 16


def _make_agg(N_table, N_pad, D, E_pad):
  NW = _NC * _NS
  EPW = E_pad // NW
  n_chunks = EPW // _CHUNK
  n_groups = n_chunks // _G
  rows_per_tile = N_pad // _NS
  zchunks = rows_per_tile // _CHUNK
  mesh = plsc.VectorSubcoreMesh(core_axis_name="c", subcore_axis_name="s")
  out_sds = jax.ShapeDtypeStruct((N_pad, D), jnp.float32)

  @functools.partial(
      pl.kernel,
      mesh=mesh,
      out_type=(out_sds, out_sds),
      scratch_types=[
          pltpu.VMEM((2, _G, _CHUNK), jnp.int32),
          pltpu.VMEM((2, _G, _CHUNK), jnp.int32),
          pltpu.VMEM((2, _CHUNK, D), jnp.float32),
          pltpu.VMEM_SHARED((N_pad, D), jnp.float32),
          pltpu.SemaphoreType.DMA,
          pltpu.SemaphoreType.DMA,
          pltpu.SemaphoreType.DMA,
          pltpu.SemaphoreType.DMA,
          pltpu.SemaphoreType.DMA,
      ],
  )
  def agg(table_hbm, src_hbm, dst_hbm, zeros_hbm, out0_hbm, out1_hbm,
          src_v, dst_v, rows_v, acc_sh, g0, g1, s0, s1, si):
    sem_g = (g0, g1)
    sem_s = (s0, s1)
    cid = lax.axis_index("c")
    sid = lax.axis_index("s")
    wid = sid * _NC + cid
    row0 = sid * rows_per_tile
    chunk0 = wid * n_chunks

    pltpu.sync_copy(src_hbm.at[pl.ds(chunk0, _G)], src_v.at[0])
    pltpu.async_copy(table_hbm.at[src_v.at[0, 0]], rows_v.at[0], sem_g[0])

    pltpu.sync_copy(zeros_hbm, rows_v.at[1])
    for z in range(zchunks):
      pltpu.async_copy(rows_v.at[1],
                       acc_sh.at[pl.ds(row0 + z * _CHUNK, _CHUNK)], s0)
    pltpu.sync_copy(dst_hbm.at[pl.ds(chunk0, _G)], dst_v.at[0])
    for z in range(zchunks):
      pltpu.make_async_copy(
          rows_v.at[1], acc_sh.at[pl.ds(row0 + z * _CHUNK, _CHUNK)],
          s0).wait()
    plsc.subcore_barrier()

    def group_body(g, carry):
      gslot = lax.rem(g, 2)
      nslot = 1 - gslot
      for k in range(_G):
        b = k % 2
        if k == 0:
          @pl.when(g >= 1)
          def _():
            pltpu.make_async_copy(
                rows_v.at[1], acc_sh.at[dst_v.at[nslot, _G - 1]],
                sem_s[1]).wait()
          @pl.when(g + 1 < n_groups)
          def _():
            nxt = chunk0 + (g + 1) * _G
            pltpu.async_copy(src_hbm.at[pl.ds(nxt, _G)], src_v.at[nslot], si)
            pltpu.async_copy(dst_hbm.at[pl.ds(nxt, _G)], dst_v.at[nslot], si)
        else:
          pltpu.make_async_copy(
              rows_v.at[1 - b], acc_sh.at[dst_v.at[gslot, k - 1]],
              sem_s[1 - b]).wait()
        if k < _G - 1:
          pltpu.async_copy(
              table_hbm.at[src_v.at[gslot, k + 1]], rows_v.at[1 - b],
              sem_g[1 - b])
        else:
          @pl.when(g + 1 < n_groups)
          def _():
            nxt = chunk0 + (g + 1) * _G
            pltpu.make_async_copy(
                src_hbm.at[pl.ds(nxt, _G)], src_v.at[nslot], si).wait()
            pltpu.make_async_copy(
                dst_hbm.at[pl.ds(nxt, _G)], dst_v.at[nslot], si).wait()
            pltpu.async_copy(
                table_hbm.at[src_v.at[nslot, 0]], rows_v.at[1 - b],
                sem_g[1 - b])
        pltpu.make_async_copy(
            table_hbm.at[src_v.at[gslot, k]], rows_v.at[b], sem_g[b]).wait()
        pltpu.async_copy(
            rows_v.at[b], acc_sh.at[dst_v.at[gslot, k]], sem_s[b], add=True)
      return carry

    lax.fori_loop(0, n_groups, group_body, 0)

    last = lax.rem(n_groups - 1, 2)
    pltpu.make_async_copy(
        rows_v.at[1], acc_sh.at[dst_v.at[last, _G - 1]], sem_s[1]).wait()
    plsc.subcore_barrier()

    def wb_in(z, b):
      pltpu.async_copy(acc_sh.at[pl.ds(row0 + z * _CHUNK, _CHUNK)],
                       rows_v.at[b], sem_g[b])

    def wb_out(z, b):
      r = row0 + z * _CHUNK

      @pl.when(cid == 0)
      def _():
        pltpu.async_copy(rows_v.at[b], out0_hbm.at[pl.ds(r, _CHUNK)],
                         sem_s[b])

      @pl.when(cid != 0)
      def _():
        pltpu.async_copy(rows_v.at[b], out1_hbm.at[pl.ds(r, _CHUNK)],
                         sem_s[b])

    def wb_wait_in(z, b):
      pltpu.make_async_copy(acc_sh.at[pl.ds(row0 + z * _CHUNK, _CHUNK)],
                            rows_v.at[b], sem_g[b]).wait()

    def wb_wait_out(z, b):
      r = row0 + z * _CHUNK

      @pl.when(cid == 0)
      def _():
        pltpu.make_async_copy(rows_v.at[b], out0_hbm.at[pl.ds(r, _CHUNK)],
                              sem_s[b]).wait()

      @pl.when(cid != 0)
      def _():
        pltpu.make_async_copy(rows_v.at[b], out1_hbm.at[pl.ds(r, _CHUNK)],
                              sem_s[b]).wait()

    wb_in(0, 0)
    for z in range(zchunks):
      b = z % 2
      wb_wait_in(z, b)
      wb_out(z, b)
      if z + 1 < zchunks:
        if z >= 1:
          wb_wait_out(z - 1, 1 - b)
        wb_in(z + 1, 1 - b)
    for z in (zchunks - 2, zchunks - 1):
      wb_wait_out(z, z % 2)

  return agg


def _dense(base, p0, p1, w_t, b, n_out):
  D = base.shape[1]
  H = w_t.shape[1]
  BM = 2000
  grid = n_out // BM

  def body(base_ref, p0_ref, p1_ref, w_ref, b_ref, o_ref):
    h = base_ref[...] + p0_ref[...] + p1_ref[...]
    acc = jnp.dot(h, w_ref[...], preferred_element_type=jnp.float32)
    acc = acc + b_ref[...]
    o_ref[...] = jnp.where(acc > 0, acc, jnp.exp(acc) - 1.0)

  return pl.pallas_call(
      body,
      grid=(grid,),
      in_specs=[
          pl.BlockSpec((BM, D), lambda i: (i, 0)),
          pl.BlockSpec((BM, D), lambda i: (i, 0)),
          pl.BlockSpec((BM, D), lambda i: (i, 0)),
          pl.BlockSpec((D, H), lambda i: (0, 0)),
          pl.BlockSpec((1, H), lambda i: (0, 0)),
      ],
      out_specs=pl.BlockSpec((BM, H), lambda i: (i, 0)),
      out_shape=jax.ShapeDtypeStruct((n_out, H), jnp.float32),
  )(base, p0, p1, w_t, b.reshape(1, H))


def kernel(features, edge_index, W1, b1, W2, b2):
  N, D = features.shape
  E = edge_index.shape[1]
  H = W1.shape[0]
  NW = _NC * _NS

  N_pad = ((N + 8 + 255) // 256) * 256
  step = NW * _CHUNK * _G
  E_pad = ((E + step - 1) // step) * step

  src, dst = edge_index[0], edge_index[1]
  pad_e = E_pad - E
  if E % NW == 0 and pad_e % NW == 0:
    ppt = pad_e // NW
    cyc = jnp.arange(ppt, dtype=jnp.int32) % (N_pad - N)
    pad_blk = jnp.broadcast_to(cyc, (NW, ppt))
    src_p = jnp.concatenate(
        [src.reshape(NW, E // NW), pad_blk], axis=1).reshape(-1, _CHUNK)
    dst_p = jnp.concatenate(
        [dst.reshape(NW, E // NW), N + pad_blk], axis=1).reshape(-1, _CHUNK)
  else:
    cyc = jnp.arange(pad_e, dtype=jnp.int32) % (N_pad - N)
    src_p = jnp.concatenate([src, cyc]).reshape(-1, _CHUNK)
    dst_p = jnp.concatenate([dst, N + cyc]).reshape(-1, _CHUNK)
  zeros_rows = jnp.zeros((_CHUNK, D), jnp.float32)

  agg1 = _make_agg(N, N_pad, D, E_pad)
  p0, p1 = agg1(features, src_p, dst_p, zeros_rows)
  x = _dense(features, p0, p1, W1.T, b1, N)

  agg2 = _make_agg(N, N_pad, H, E_pad)
  q0, q1 = agg2(x, src_p, dst_p, zeros_rows)
  return _dense(x, q0, q1, W2.T, b2, N)
